# baseline (speedup 1.0000x reference)
import functools

import jax
import jax.numpy as jnp
from jax import lax
from jax.experimental import pallas as pl
from jax.experimental.pallas import tpu as pltpu
from jax.experimental.pallas import tpu_sc as plsc

NS = 16
LANES = 16
CHUNK = 128


def _t1_body(x_ref, w0_ref, b0_ref, wa_ref, wb_ref, o0_ref, oa_ref, ob_ref):
    x = x_ref[...]
    o0_ref[...] = jnp.dot(x, w0_ref[...], preferred_element_type=jnp.float32) + b0_ref[...]
    oa_ref[...] = jnp.dot(x, wa_ref[...], preferred_element_type=jnp.float32)
    ob_ref[...] = jnp.dot(x, wb_ref[...], preferred_element_type=jnp.float32)


def _t1(x, W0, b0, Wa, Wb):
    n, d = x.shape
    h = W0.shape[1]
    blk = 2000
    grid = n // blk
    return pl.pallas_call(
        _t1_body,
        grid=(grid,),
        in_specs=[
            pl.BlockSpec((blk, d), lambda i: (i, 0)),
            pl.BlockSpec((d, h), lambda i: (0, 0)),
            pl.BlockSpec((1, h), lambda i: (0, 0)),
            pl.BlockSpec((d, h), lambda i: (0, 0)),
            pl.BlockSpec((d, h), lambda i: (0, 0)),
        ],
        out_specs=[
            pl.BlockSpec((blk, h), lambda i: (i, 0)),
            pl.BlockSpec((blk, h), lambda i: (i, 0)),
            pl.BlockSpec((blk, h), lambda i: (i, 0)),
        ],
        out_shape=[
            jax.ShapeDtypeStruct((n, h), jnp.float32),
            jax.ShapeDtypeStruct((n, h), jnp.float32),
            jax.ShapeDtypeStruct((n, h), jnp.float32),
        ],
    )(x, W0, b0.reshape(1, h), Wa, Wb)


def _bn_mm3_body(x0_ref, pa_ref, pb_ref, bsum_ref, g_ref, bb_ref,
                 w0_ref, b0_ref, wa_ref, wb_ref, scale_ref,
                 o0_ref, oa_ref, ob_ref):
    left = x0_ref[...]
    nrows = left.shape[0]
    right = scale_ref[0] * (pa_ref[...][:nrows] + pb_ref[...][:nrows] + bsum_ref[...])
    u = jnp.concatenate([left, right], axis=1)
    mu = jnp.sum(u, axis=0, keepdims=True) / nrows
    xc = u - mu
    var = jnp.sum(xc * xc, axis=0, keepdims=True) / nrows
    z = xc * lax.rsqrt(var + 1e-5) * g_ref[...] + bb_ref[...]
    o0_ref[...] = jnp.dot(z, w0_ref[...], preferred_element_type=jnp.float32) + b0_ref[...]
    oa_ref[...] = jnp.dot(z, wa_ref[...], preferred_element_type=jnp.float32)
    ob_ref[...] = jnp.dot(z, wb_ref[...], preferred_element_type=jnp.float32)


def _t2(x0, pa, pb, bsum, scale, g, bb, W0, b0, Wa, Wb):
    n, h = x0.shape
    c = W0.shape[1]
    return pl.pallas_call(
        _bn_mm3_body,
        out_shape=[
            jax.ShapeDtypeStruct((n, c), jnp.float32),
            jax.ShapeDtypeStruct((n, c), jnp.float32),
            jax.ShapeDtypeStruct((n, c), jnp.float32),
        ],
    )(x0, pa, pb, bsum.reshape(1, h), g.reshape(1, 2 * h), bb.reshape(1, 2 * h),
      W0, b0.reshape(1, c), Wa, Wb, jnp.asarray([scale], jnp.float32))


def _bn_final_body(x0_ref, pa_ref, pb_ref, bsum_ref, g_ref, bb_ref,
                   wt_ref, bout_ref, scale_ref, o_ref):
    left = x0_ref[...]
    nrows = left.shape[0]
    right = scale_ref[0] * (pa_ref[...][:nrows] + pb_ref[...][:nrows] + bsum_ref[...])
    u = jnp.concatenate([left, right], axis=1)
    mu = jnp.sum(u, axis=0, keepdims=True) / nrows
    xc = u - mu
    var = jnp.sum(xc * xc, axis=0, keepdims=True) / nrows
    z = xc * lax.rsqrt(var + 1e-5) * g_ref[...] + bb_ref[...]
    o_ref[...] = jnp.dot(z, wt_ref[...], preferred_element_type=jnp.float32) + bout_ref[...]


def _t3(x0, pa, pb, bsum, scale, g, bb, Wt, bout):
    n, c = x0.shape
    cout = Wt.shape[1]
    return pl.pallas_call(
        _bn_final_body,
        out_shape=jax.ShapeDtypeStruct((n, cout), jnp.float32),
    )(x0, pa, pb, bsum.reshape(1, c), g.reshape(1, 2 * c), bb.reshape(1, 2 * c),
      Wt, bout.reshape(1, cout), jnp.asarray([scale], jnp.float32))


def _sc_conv_pair(hA, srcA, dstA, wA, hB, srcB, dstB, wB):
    n, f = hA.shape
    n_chunks = srcA.shape[0] // NS
    n_acc = -(-n // (NS * 8)) * (NS * 8)
    rows_per_tile = n_acc // NS
    mesh = plsc.VectorSubcoreMesh(core_axis_name="c", subcore_axis_name="s")

    @functools.partial(
        pl.kernel,
        out_type=(
            jax.ShapeDtypeStruct((n_acc, f), jnp.float32),
            jax.ShapeDtypeStruct((n_acc, f), jnp.float32),
        ),
        mesh=mesh,
        compiler_params=pltpu.CompilerParams(use_tc_tiling_on_sc=False),
        scratch_types=[
            pltpu.VMEM((n_chunks, CHUNK), jnp.int32),
            pltpu.VMEM((n_chunks, CHUNK), jnp.int32),
            pltpu.VMEM((CHUNK, f), jnp.float32),
            pltpu.VMEM((CHUNK, f), jnp.float32),
            pltpu.VMEM((CHUNK, LANES), jnp.float32),
            pltpu.VMEM((CHUNK, LANES), jnp.float32),
            pltpu.VMEM_SHARED((n_acc, f), jnp.float32),
            pltpu.SemaphoreType.DMA,
            pltpu.SemaphoreType.DMA,
            pltpu.SemaphoreType.DMA,
            pltpu.SemaphoreType.DMA,
        ],
    )
    def k(hA_ref, srcA_ref, dstA_ref, wA_ref, hB_ref, srcB_ref, dstB_ref, wB_ref,
          outA_ref, outB_ref, src_v, dst_v, rows_a, rows_b, wv_a, wv_b, acc,
          sem_ra, sem_rb, sem_wa, sem_wb):
        c = lax.axis_index("c")
        s = lax.axis_index("s")
        zero16 = jnp.zeros((LANES,), jnp.float32)

        def run(h_ref, src_ref, dst_ref, w_ref, out_ref):
            base = s * n_chunks
            pltpu.sync_copy(src_ref.at[pl.ds(base, n_chunks)], src_v)
            pltpu.sync_copy(dst_ref.at[pl.ds(base, n_chunks)], dst_v)

            def zrow(i, carry):
                for g in range(f // LANES):
                    rows_a[i, pl.ds(g * LANES, LANES)] = zero16
                return carry
            lax.fori_loop(0, CHUNK, zrow, 0)
            row0 = s * rows_per_tile
            nfull = rows_per_tile // CHUNK
            for blk in range(nfull):
                pltpu.sync_copy(rows_a, acc.at[pl.ds(row0 + blk * CHUNK, CHUNK)])
            rem = rows_per_tile - nfull * CHUNK
            if rem:
                pltpu.sync_copy(rows_a.at[pl.ds(0, rem)],
                                acc.at[pl.ds(row0 + nfull * CHUNK, rem)])
            plsc.subcore_barrier()

            ebase = base * CHUNK

            def gather(j, rows, wbuf, sem_r, sem_w):
                pltpu.async_copy(h_ref.at[src_v.at[j]], rows, sem_r)
                pltpu.async_copy(w_ref.at[pl.ds(ebase + j * CHUNK, CHUNK)],
                                 wbuf, sem_w)

            def wait_gather(rows, wbuf, sem_r, sem_w):
                pltpu.make_async_copy(h_ref.at[src_v.at[0]], rows, sem_r).wait()
                pltpu.make_async_copy(w_ref.at[pl.ds(0, CHUNK)], wbuf, sem_w).wait()

            def compute(rows, wbuf):
                def edge_body(e, c2):
                    wv = wbuf[e, :]
                    for g in range(f // LANES):
                        sl = rows[e, pl.ds(g * LANES, LANES)]
                        rows[e, pl.ds(g * LANES, LANES)] = sl * wv
                    return c2
                lax.fori_loop(0, CHUNK, edge_body, 0, unroll=4)

            gather(0, rows_a, wv_a, sem_ra, sem_wa)

            def pair_body(p, carry):
                j0 = 2 * p
                j1 = j0 + 1
                gather(j1, rows_b, wv_b, sem_rb, sem_wb)
                wait_gather(rows_a, wv_a, sem_ra, sem_wa)
                compute(rows_a, wv_a)
                pltpu.sync_copy(rows_a, acc.at[dst_v.at[j0]], add=True)

                @pl.when(j0 + 2 < n_chunks)
                def _():
                    gather(j0 + 2, rows_a, wv_a, sem_ra, sem_wa)
                wait_gather(rows_b, wv_b, sem_rb, sem_wb)
                compute(rows_b, wv_b)
                pltpu.sync_copy(rows_b, acc.at[dst_v.at[j1]], add=True)
                return carry
            lax.fori_loop(0, n_chunks // 2, pair_body, 0)
            plsc.subcore_barrier()
            pltpu.sync_copy(acc.at[pl.ds(row0, rows_per_tile)],
                            out_ref.at[pl.ds(row0, rows_per_tile)])

        @pl.when(c == 0)
        def _():
            run(hA_ref, srcA_ref, dstA_ref, wA_ref, outA_ref)

        @pl.when(c == 1)
        def _():
            run(hB_ref, srcB_ref, dstB_ref, wB_ref, outB_ref)

    return k(hA, srcA, dstA, wA, hB, srcB, dstB, wB)


def _prep_edges(ei, ew):
    e = ei.shape[1]
    n_chunks = -(-e // (NS * CHUNK))
    n_chunks = -(-n_chunks // 8) * 8
    epad = NS * n_chunks * CHUNK
    pad = epad - e
    src = jnp.concatenate([ei[0], jnp.zeros((pad,), jnp.int32)]).reshape(NS * n_chunks, CHUNK)
    dst = jnp.concatenate([ei[1], jnp.zeros((pad,), jnp.int32)]).reshape(NS * n_chunks, CHUNK)
    w = jnp.concatenate([ew, jnp.zeros((pad,), jnp.float32)])
    wexp = jnp.broadcast_to(w[:, None], (epad, LANES))
    return src, dst, wexp


def kernel(features, edge_index, edge_weight, edge_index2, edge_weight2,
           W_ln1, b_ln1, W_c1a, b_c1a, W_c1b, b_c1b, bn1_g, bn1_b,
           W_ln2, b_ln2, W_c2a, b_c2a, W_c2b, b_c2b, bn2_g, bn2_b,
           W_conv, b_conv):
    srcA, dstA, wA = _prep_edges(edge_index, edge_weight)
    srcB, dstB, wB = _prep_edges(edge_index2, edge_weight2)

    x0, ha, hb = _t1(features, W_ln1, b_ln1, W_c1a, W_c1b)
    pA, pB = _sc_conv_pair(ha, srcA, dstA, wA, hb, srcB, dstB, wB)
    y0, h2a, h2b = _t2(x0, pA, pB, b_c1a + b_c1b, 2.0, bn1_g, bn1_b,
                       W_ln2, b_ln2, W_c2a, W_c2b)
    qA, qB = _sc_conv_pair(h2a, srcA, dstA, wA, h2b, srcB, dstB, wB)
    out = _t3(y0, qA, qB, b_c2a + b_c2b, 0.5, bn2_g, bn2_b,
              W_conv.T, b_conv)
    return out


# speedup vs baseline: 1.0296x; 1.0296x over previous
import functools

import jax
import jax.numpy as jnp
from jax import lax
from jax.experimental import pallas as pl
from jax.experimental.pallas import tpu as pltpu
from jax.experimental.pallas import tpu_sc as plsc

NS = 16
LANES = 16
CHUNK = 128


def _t1_body(x_ref, w0_ref, b0_ref, wa_ref, wb_ref, o0_ref, oa_ref, ob_ref):
    x = x_ref[...]
    o0_ref[...] = jnp.dot(x, w0_ref[...], preferred_element_type=jnp.float32) + b0_ref[...]
    oa_ref[...] = jnp.dot(x, wa_ref[...], preferred_element_type=jnp.float32)
    ob_ref[...] = jnp.dot(x, wb_ref[...], preferred_element_type=jnp.float32)


def _t1(x, W0, b0, Wa, Wb):
    n, d = x.shape
    h = W0.shape[1]
    blk = 2000
    grid = n // blk
    return pl.pallas_call(
        _t1_body,
        grid=(grid,),
        in_specs=[
            pl.BlockSpec((blk, d), lambda i: (i, 0)),
            pl.BlockSpec((d, h), lambda i: (0, 0)),
            pl.BlockSpec((1, h), lambda i: (0, 0)),
            pl.BlockSpec((d, h), lambda i: (0, 0)),
            pl.BlockSpec((d, h), lambda i: (0, 0)),
        ],
        out_specs=[
            pl.BlockSpec((blk, h), lambda i: (i, 0)),
            pl.BlockSpec((blk, h), lambda i: (i, 0)),
            pl.BlockSpec((blk, h), lambda i: (i, 0)),
        ],
        out_shape=[
            jax.ShapeDtypeStruct((n, h), jnp.float32),
            jax.ShapeDtypeStruct((n, h), jnp.float32),
            jax.ShapeDtypeStruct((n, h), jnp.float32),
        ],
    )(x, W0, b0.reshape(1, h), Wa, Wb)


def _bn_mm3_body(x0_ref, pa_ref, pb_ref, bsum_ref, g_ref, bb_ref,
                 w0_ref, b0_ref, wa_ref, wb_ref, scale_ref,
                 o0_ref, oa_ref, ob_ref):
    left = x0_ref[...]
    nrows = left.shape[0]
    right = scale_ref[0] * (pa_ref[...][:nrows] + pb_ref[...][:nrows] + bsum_ref[...])
    u = jnp.concatenate([left, right], axis=1)
    mu = jnp.sum(u, axis=0, keepdims=True) / nrows
    xc = u - mu
    var = jnp.sum(xc * xc, axis=0, keepdims=True) / nrows
    z = xc * lax.rsqrt(var + 1e-5) * g_ref[...] + bb_ref[...]
    o0_ref[...] = jnp.dot(z, w0_ref[...], preferred_element_type=jnp.float32) + b0_ref[...]
    oa_ref[...] = jnp.dot(z, wa_ref[...], preferred_element_type=jnp.float32)
    ob_ref[...] = jnp.dot(z, wb_ref[...], preferred_element_type=jnp.float32)


def _t2(x0, pa, pb, bsum, scale, g, bb, W0, b0, Wa, Wb):
    n, h = x0.shape
    c = W0.shape[1]
    return pl.pallas_call(
        _bn_mm3_body,
        out_shape=[
            jax.ShapeDtypeStruct((n, c), jnp.float32),
            jax.ShapeDtypeStruct((n, c), jnp.float32),
            jax.ShapeDtypeStruct((n, c), jnp.float32),
        ],
    )(x0, pa, pb, bsum.reshape(1, h), g.reshape(1, 2 * h), bb.reshape(1, 2 * h),
      W0, b0.reshape(1, c), Wa, Wb, jnp.asarray([scale], jnp.float32))


def _bn_final_body(x0_ref, pa_ref, pb_ref, bsum_ref, g_ref, bb_ref,
                   wt_ref, bout_ref, scale_ref, o_ref):
    left = x0_ref[...]
    nrows = left.shape[0]
    right = scale_ref[0] * (pa_ref[...][:nrows] + pb_ref[...][:nrows] + bsum_ref[...])
    u = jnp.concatenate([left, right], axis=1)
    mu = jnp.sum(u, axis=0, keepdims=True) / nrows
    xc = u - mu
    var = jnp.sum(xc * xc, axis=0, keepdims=True) / nrows
    z = xc * lax.rsqrt(var + 1e-5) * g_ref[...] + bb_ref[...]
    o_ref[...] = jnp.dot(z, wt_ref[...], preferred_element_type=jnp.float32) + bout_ref[...]


def _t3(x0, pa, pb, bsum, scale, g, bb, Wt, bout):
    n, c = x0.shape
    cout = Wt.shape[1]
    return pl.pallas_call(
        _bn_final_body,
        out_shape=jax.ShapeDtypeStruct((n, cout), jnp.float32),
    )(x0, pa, pb, bsum.reshape(1, c), g.reshape(1, 2 * c), bb.reshape(1, 2 * c),
      Wt, bout.reshape(1, cout), jnp.asarray([scale], jnp.float32))


def _sc_conv_pair(hA, srcA, dstA, wA, hB, srcB, dstB, wB, stage):
    n, f = hA.shape
    n_chunks = srcA.shape[0] // NS
    n_acc = n
    rows_per_tile = n_acc // NS
    mesh = plsc.VectorSubcoreMesh(core_axis_name="c", subcore_axis_name="s")

    @functools.partial(
        pl.kernel,
        out_type=(
            jax.ShapeDtypeStruct((n_acc, f), jnp.float32),
            jax.ShapeDtypeStruct((n_acc, f), jnp.float32),
        ),
        mesh=mesh,
        compiler_params=pltpu.CompilerParams(use_tc_tiling_on_sc=False),
        scratch_types=[
            pltpu.VMEM((n_chunks, CHUNK), jnp.int32),
            pltpu.VMEM((n_chunks, CHUNK), jnp.int32),
            pltpu.VMEM((CHUNK, f), jnp.float32),
            pltpu.VMEM((CHUNK, f), jnp.float32),
            pltpu.VMEM((CHUNK, LANES), jnp.float32),
            pltpu.VMEM((CHUNK, LANES), jnp.float32),
            pltpu.VMEM_SHARED((n_acc, f), jnp.float32),
            pltpu.VMEM_SHARED((n if stage else 1, f), jnp.float32),
            pltpu.SemaphoreType.DMA,
            pltpu.SemaphoreType.DMA,
            pltpu.SemaphoreType.DMA,
            pltpu.SemaphoreType.DMA,
        ],
    )
    def k(hA_ref, srcA_ref, dstA_ref, wA_ref, hB_ref, srcB_ref, dstB_ref, wB_ref,
          outA_ref, outB_ref, src_v, dst_v, rows_a, rows_b, wv_a, wv_b, acc, h_s,
          sem_ra, sem_rb, sem_wa, sem_wb):
        c = lax.axis_index("c")
        s = lax.axis_index("s")
        zero16 = jnp.zeros((LANES,), jnp.float32)

        def run(h_ref, src_ref, dst_ref, w_ref, out_ref):
            base = s * n_chunks
            if stage:
                stage_rows = n // NS
                nfs = stage_rows // CHUNK
                for b in range(nfs):
                    off = s * stage_rows + b * CHUNK
                    pltpu.sync_copy(h_ref.at[pl.ds(off, CHUNK)], rows_a)
                    pltpu.sync_copy(rows_a, h_s.at[pl.ds(off, CHUNK)])
                rem_s = stage_rows - nfs * CHUNK
                if rem_s:
                    off = s * stage_rows + nfs * CHUNK
                    pltpu.sync_copy(h_ref.at[pl.ds(off, rem_s)],
                                    rows_a.at[pl.ds(0, rem_s)])
                    pltpu.sync_copy(rows_a.at[pl.ds(0, rem_s)],
                                    h_s.at[pl.ds(off, rem_s)])
            h_src = h_s if stage else h_ref
            pltpu.sync_copy(src_ref.at[pl.ds(base, n_chunks)], src_v)
            pltpu.sync_copy(dst_ref.at[pl.ds(base, n_chunks)], dst_v)

            def zrow(i, carry):
                for g in range(f // LANES):
                    rows_a[i, pl.ds(g * LANES, LANES)] = zero16
                return carry
            lax.fori_loop(0, CHUNK, zrow, 0)
            row0 = s * rows_per_tile
            nfull = rows_per_tile // CHUNK
            for blk in range(nfull):
                pltpu.sync_copy(rows_a, acc.at[pl.ds(row0 + blk * CHUNK, CHUNK)])
            rem = rows_per_tile - nfull * CHUNK
            if rem:
                pltpu.sync_copy(rows_a.at[pl.ds(0, rem)],
                                acc.at[pl.ds(row0 + nfull * CHUNK, rem)])
            plsc.subcore_barrier()

            ebase = base * CHUNK

            def gather(j, rows, wbuf, sem_r, sem_w):
                pltpu.async_copy(h_src.at[src_v.at[j]], rows, sem_r)
                pltpu.async_copy(w_ref.at[pl.ds(ebase + j * CHUNK, CHUNK)],
                                 wbuf, sem_w)

            def wait_gather(rows, wbuf, sem_r, sem_w):
                pltpu.make_async_copy(h_src.at[src_v.at[0]], rows, sem_r).wait()
                pltpu.make_async_copy(w_ref.at[pl.ds(0, CHUNK)], wbuf, sem_w).wait()

            def compute(rows, wbuf):
                def edge_body(e, c2):
                    wv = wbuf[e, :]
                    for g in range(f // LANES):
                        sl = rows[e, pl.ds(g * LANES, LANES)]
                        rows[e, pl.ds(g * LANES, LANES)] = sl * wv
                    return c2
                lax.fori_loop(0, CHUNK, edge_body, 0, unroll=4)

            gather(0, rows_a, wv_a, sem_ra, sem_wa)

            def pair_body(p, carry):
                j0 = 2 * p
                j1 = j0 + 1
                gather(j1, rows_b, wv_b, sem_rb, sem_wb)
                wait_gather(rows_a, wv_a, sem_ra, sem_wa)
                compute(rows_a, wv_a)
                pltpu.sync_copy(rows_a, acc.at[dst_v.at[j0]], add=True)

                @pl.when(j0 + 2 < n_chunks)
                def _():
                    gather(j0 + 2, rows_a, wv_a, sem_ra, sem_wa)
                wait_gather(rows_b, wv_b, sem_rb, sem_wb)
                compute(rows_b, wv_b)
                pltpu.sync_copy(rows_b, acc.at[dst_v.at[j1]], add=True)
                return carry
            lax.fori_loop(0, n_chunks // 2, pair_body, 0)
            plsc.subcore_barrier()
            pltpu.sync_copy(acc.at[pl.ds(row0, rows_per_tile)],
                            out_ref.at[pl.ds(row0, rows_per_tile)])

        @pl.when(c == 0)
        def _():
            run(hA_ref, srcA_ref, dstA_ref, wA_ref, outA_ref)

        @pl.when(c == 1)
        def _():
            run(hB_ref, srcB_ref, dstB_ref, wB_ref, outB_ref)

    return k(hA, srcA, dstA, wA, hB, srcB, dstB, wB)


def _prep_edges(ei, ew):
    e = ei.shape[1]
    n_chunks = -(-e // (NS * CHUNK))
    n_chunks = -(-n_chunks // 8) * 8
    epad = NS * n_chunks * CHUNK
    pad = epad - e
    src = jnp.concatenate([ei[0], jnp.zeros((pad,), jnp.int32)]).reshape(NS * n_chunks, CHUNK)
    dst = jnp.concatenate([ei[1], jnp.zeros((pad,), jnp.int32)]).reshape(NS * n_chunks, CHUNK)
    w = jnp.concatenate([ew, jnp.zeros((pad,), jnp.float32)])
    wexp = jnp.broadcast_to(w[:, None], (epad, LANES))
    return src, dst, wexp


def kernel(features, edge_index, edge_weight, edge_index2, edge_weight2,
           W_ln1, b_ln1, W_c1a, b_c1a, W_c1b, b_c1b, bn1_g, bn1_b,
           W_ln2, b_ln2, W_c2a, b_c2a, W_c2b, b_c2b, bn2_g, bn2_b,
           W_conv, b_conv):
    srcA, dstA, wA = _prep_edges(edge_index, edge_weight)
    srcB, dstB, wB = _prep_edges(edge_index2, edge_weight2)

    x0, ha, hb = _t1(features, W_ln1, b_ln1, W_c1a, W_c1b)
    hf = ha.shape[1] // 2
    pA_lo, pB_lo = _sc_conv_pair(ha[:, :hf], srcA, dstA, wA,
                                 hb[:, :hf], srcB, dstB, wB, stage=True)
    pA_hi, pB_hi = _sc_conv_pair(ha[:, hf:], srcA, dstA, wA,
                                 hb[:, hf:], srcB, dstB, wB, stage=True)
    pA = jnp.concatenate([pA_lo, pA_hi], axis=1)
    pB = jnp.concatenate([pB_lo, pB_hi], axis=1)
    y0, h2a, h2b = _t2(x0, pA, pB, b_c1a + b_c1b, 2.0, bn1_g, bn1_b,
                       W_ln2, b_ln2, W_c2a, W_c2b)
    qA, qB = _sc_conv_pair(h2a, srcA, dstA, wA, h2b, srcB, dstB, wB, stage=True)
    out = _t3(y0, qA, qB, b_c2a + b_c2b, 0.5, bn2_g, bn2_b,
              W_conv.T, b_conv)
    return out


# speedup vs baseline: 2.8674x; 2.7849x over previous
import functools

import jax
import jax.numpy as jnp
from jax import lax
from jax.experimental import pallas as pl
from jax.experimental.pallas import tpu as pltpu
from jax.experimental.pallas import tpu_sc as plsc

NS = 16
LANES = 16
CHUNK = 128


def _t1_body(x_ref, w0_ref, b0_ref, wa_ref, wb_ref, o0_ref, oa_ref, ob_ref):
    x = x_ref[...]
    o0_ref[...] = jnp.dot(x, w0_ref[...], preferred_element_type=jnp.float32) + b0_ref[...]
    oa_ref[...] = jnp.dot(x, wa_ref[...], preferred_element_type=jnp.float32)
    ob_ref[...] = jnp.dot(x, wb_ref[...], preferred_element_type=jnp.float32)


def _t1(x, W0, b0, Wa, Wb):
    n, d = x.shape
    h = W0.shape[1]
    blk = 2000
    grid = n // blk
    return pl.pallas_call(
        _t1_body,
        grid=(grid,),
        in_specs=[
            pl.BlockSpec((blk, d), lambda i: (i, 0)),
            pl.BlockSpec((d, h), lambda i: (0, 0)),
            pl.BlockSpec((1, h), lambda i: (0, 0)),
            pl.BlockSpec((d, h), lambda i: (0, 0)),
            pl.BlockSpec((d, h), lambda i: (0, 0)),
        ],
        out_specs=[
            pl.BlockSpec((blk, h), lambda i: (i, 0)),
            pl.BlockSpec((blk, h), lambda i: (i, 0)),
            pl.BlockSpec((blk, h), lambda i: (i, 0)),
        ],
        out_shape=[
            jax.ShapeDtypeStruct((n, h), jnp.float32),
            jax.ShapeDtypeStruct((n, h), jnp.float32),
            jax.ShapeDtypeStruct((n, h), jnp.float32),
        ],
    )(x, W0, b0.reshape(1, h), Wa, Wb)


def _bn_mm3_body(x0_ref, pa_ref, pb_ref, bsum_ref, g_ref, bb_ref,
                 w0_ref, b0_ref, wa_ref, wb_ref, scale_ref,
                 o0_ref, oa_ref, ob_ref):
    left = x0_ref[...]
    nrows = left.shape[0]
    right = scale_ref[0] * (pa_ref[...][:nrows] + pb_ref[...][:nrows] + bsum_ref[...])
    u = jnp.concatenate([left, right], axis=1)
    mu = jnp.sum(u, axis=0, keepdims=True) / nrows
    xc = u - mu
    var = jnp.sum(xc * xc, axis=0, keepdims=True) / nrows
    z = xc * lax.rsqrt(var + 1e-5) * g_ref[...] + bb_ref[...]
    o0_ref[...] = jnp.dot(z, w0_ref[...], preferred_element_type=jnp.float32) + b0_ref[...]
    oa_ref[...] = jnp.dot(z, wa_ref[...], preferred_element_type=jnp.float32)
    ob_ref[...] = jnp.dot(z, wb_ref[...], preferred_element_type=jnp.float32)


def _t2(x0, pa, pb, bsum, scale, g, bb, W0, b0, Wa, Wb):
    n, h = x0.shape
    c = W0.shape[1]
    return pl.pallas_call(
        _bn_mm3_body,
        out_shape=[
            jax.ShapeDtypeStruct((n, c), jnp.float32),
            jax.ShapeDtypeStruct((n, c), jnp.float32),
            jax.ShapeDtypeStruct((n, c), jnp.float32),
        ],
    )(x0, pa, pb, bsum.reshape(1, h), g.reshape(1, 2 * h), bb.reshape(1, 2 * h),
      W0, b0.reshape(1, c), Wa, Wb, jnp.asarray([scale], jnp.float32))


def _bn_final_body(x0_ref, pa_ref, pb_ref, bsum_ref, g_ref, bb_ref,
                   wt_ref, bout_ref, scale_ref, o_ref):
    left = x0_ref[...]
    nrows = left.shape[0]
    right = scale_ref[0] * (pa_ref[...][:nrows] + pb_ref[...][:nrows] + bsum_ref[...])
    u = jnp.concatenate([left, right], axis=1)
    mu = jnp.sum(u, axis=0, keepdims=True) / nrows
    xc = u - mu
    var = jnp.sum(xc * xc, axis=0, keepdims=True) / nrows
    z = xc * lax.rsqrt(var + 1e-5) * g_ref[...] + bb_ref[...]
    o_ref[...] = jnp.dot(z, wt_ref[...], preferred_element_type=jnp.float32) + bout_ref[...]


def _t3(x0, pa, pb, bsum, scale, g, bb, Wt, bout):
    n, c = x0.shape
    cout = Wt.shape[1]
    return pl.pallas_call(
        _bn_final_body,
        out_shape=jax.ShapeDtypeStruct((n, cout), jnp.float32),
    )(x0, pa, pb, bsum.reshape(1, c), g.reshape(1, 2 * c), bb.reshape(1, 2 * c),
      Wt, bout.reshape(1, cout), jnp.asarray([scale], jnp.float32))


def _sc_conv_pair(hA, srcA, dstA, wA, hB, srcB, dstB, wB, stage):
    n, f = hA.shape
    n_chunks = srcA.shape[0] // NS
    n_acc = n
    rows_per_tile = n_acc // NS
    mesh = plsc.VectorSubcoreMesh(core_axis_name="c", subcore_axis_name="s")

    @functools.partial(
        pl.kernel,
        out_type=(
            jax.ShapeDtypeStruct((n_acc, f), jnp.float32),
            jax.ShapeDtypeStruct((n_acc, f), jnp.float32),
        ),
        mesh=mesh,
        compiler_params=pltpu.CompilerParams(use_tc_tiling_on_sc=False),
        scratch_types=[
            pltpu.VMEM((n_chunks, CHUNK), jnp.int32),
            pltpu.VMEM((n_chunks, CHUNK), jnp.int32),
            pltpu.VMEM((n_chunks * CHUNK,), jnp.float32),
            pltpu.VMEM((CHUNK, f), jnp.float32),
            pltpu.VMEM((CHUNK, f), jnp.float32),
            pltpu.VMEM((CHUNK, f), jnp.float32),
            pltpu.VMEM((CHUNK, f), jnp.float32),
            pltpu.VMEM_SHARED((n_acc, f), jnp.float32),
            pltpu.VMEM_SHARED((n if stage else 1, f), jnp.float32),
            pltpu.SemaphoreType.DMA,
            pltpu.SemaphoreType.DMA,
            pltpu.SemaphoreType.DMA,
            pltpu.SemaphoreType.DMA,
            pltpu.SemaphoreType.DMA,
            pltpu.SemaphoreType.DMA,
            pltpu.SemaphoreType.DMA,
            pltpu.SemaphoreType.DMA,
        ],
    )
    def k(hA_ref, srcA_ref, dstA_ref, wA_ref, hB_ref, srcB_ref, dstB_ref, wB_ref,
          outA_ref, outB_ref, src_v, dst_v, w_v, rows0, rows1, rows2, rows3,
          acc, h_s, sg0, sg1, sg2, sg3, ss0, ss1, ss2, ss3):
        rows_bufs = (rows0, rows1, rows2, rows3)
        gsems = (sg0, sg1, sg2, sg3)
        ssems = (ss0, ss1, ss2, ss3)
        c = lax.axis_index("c")
        s = lax.axis_index("s")
        zero16 = jnp.zeros((LANES,), jnp.float32)

        def run(h_ref, src_ref, dst_ref, w_ref, out_ref):
            base = s * n_chunks
            if stage:
                stage_rows = n // NS
                nfs = stage_rows // CHUNK
                for b in range(nfs):
                    off = s * stage_rows + b * CHUNK
                    pltpu.sync_copy(h_ref.at[pl.ds(off, CHUNK)], rows0)
                    pltpu.sync_copy(rows0, h_s.at[pl.ds(off, CHUNK)])
                rem_s = stage_rows - nfs * CHUNK
                if rem_s:
                    off = s * stage_rows + nfs * CHUNK
                    pltpu.sync_copy(h_ref.at[pl.ds(off, rem_s)],
                                    rows0.at[pl.ds(0, rem_s)])
                    pltpu.sync_copy(rows0.at[pl.ds(0, rem_s)],
                                    h_s.at[pl.ds(off, rem_s)])
            h_src = h_s if stage else h_ref
            pltpu.sync_copy(src_ref.at[pl.ds(base, n_chunks)], src_v)
            pltpu.sync_copy(dst_ref.at[pl.ds(base, n_chunks)], dst_v)
            pltpu.sync_copy(w_ref.at[pl.ds(base * CHUNK, n_chunks * CHUNK)], w_v)

            def zrow(i, carry):
                for g in range(f // LANES):
                    rows0[i, pl.ds(g * LANES, LANES)] = zero16
                return carry
            lax.fori_loop(0, CHUNK, zrow, 0)
            row0 = s * rows_per_tile
            nfull = rows_per_tile // CHUNK
            for blk in range(nfull):
                pltpu.sync_copy(rows0, acc.at[pl.ds(row0 + blk * CHUNK, CHUNK)])
            rem = rows_per_tile - nfull * CHUNK
            if rem:
                pltpu.sync_copy(rows0.at[pl.ds(0, rem)],
                                acc.at[pl.ds(row0 + nfull * CHUNK, rem)])
            plsc.subcore_barrier()

            def gather_start(j, b):
                pltpu.async_copy(h_src.at[src_v.at[j]], rows_bufs[b], gsems[b])

            def gather_wait(b):
                pltpu.make_async_copy(h_src.at[src_v.at[0]], rows_bufs[b],
                                      gsems[b]).wait()

            def scatter_start(j, b):
                pltpu.async_copy(rows_bufs[b], acc.at[dst_v.at[j]], ssems[b],
                                 add=True)

            def scatter_wait(b):
                pltpu.make_async_copy(rows_bufs[b], acc.at[dst_v.at[0]],
                                      ssems[b]).wait()

            def compute(j, rows):
                def group_body(ge, c2):
                    wg = w_v[pl.ds(j * CHUNK + ge * LANES, LANES)]
                    for i in range(LANES):
                        wv = jnp.full((LANES,), wg[i])
                        e = ge * LANES + i
                        for g in range(f // LANES):
                            sl = rows[e, pl.ds(g * LANES, LANES)]
                            rows[e, pl.ds(g * LANES, LANES)] = sl * wv
                    return c2
                lax.fori_loop(0, CHUNK // LANES, group_body, 0)

            gather_start(0, 0)
            gather_start(1, 1)

            def quad_body(q, carry):
                for b in range(4):
                    j = 4 * q + b
                    gather_wait(b)
                    compute(j, rows_bufs[b])
                    scatter_start(j, b)
                    b2 = (b + 2) % 4

                    @pl.when(j + 2 < n_chunks)
                    def _():
                        @pl.when(j >= 2)
                        def _():
                            scatter_wait(b2)
                        gather_start(j + 2, b2)
                return carry
            lax.fori_loop(0, n_chunks // 4, quad_body, 0)
            for b in range(4):
                scatter_wait((n_chunks - 4 + b) % 4)
            plsc.subcore_barrier()
            pltpu.sync_copy(acc.at[pl.ds(row0, rows_per_tile)],
                            out_ref.at[pl.ds(row0, rows_per_tile)])

        @pl.when(c == 0)
        def _():
            run(hA_ref, srcA_ref, dstA_ref, wA_ref, outA_ref)

        @pl.when(c == 1)
        def _():
            run(hB_ref, srcB_ref, dstB_ref, wB_ref, outB_ref)

    return k(hA, srcA, dstA, wA, hB, srcB, dstB, wB)


def _prep_edges(ei, ew):
    e = ei.shape[1]
    n_chunks = -(-e // (NS * CHUNK))
    n_chunks = -(-n_chunks // 8) * 8
    epad = NS * n_chunks * CHUNK
    pad = epad - e
    src = jnp.concatenate([ei[0], jnp.zeros((pad,), jnp.int32)]).reshape(NS * n_chunks, CHUNK)
    dst = jnp.concatenate([ei[1], jnp.zeros((pad,), jnp.int32)]).reshape(NS * n_chunks, CHUNK)
    w = jnp.concatenate([ew, jnp.zeros((pad,), jnp.float32)])
    return src, dst, w


def kernel(features, edge_index, edge_weight, edge_index2, edge_weight2,
           W_ln1, b_ln1, W_c1a, b_c1a, W_c1b, b_c1b, bn1_g, bn1_b,
           W_ln2, b_ln2, W_c2a, b_c2a, W_c2b, b_c2b, bn2_g, bn2_b,
           W_conv, b_conv):
    srcA, dstA, wA = _prep_edges(edge_index, edge_weight)
    srcB, dstB, wB = _prep_edges(edge_index2, edge_weight2)

    x0, ha, hb = _t1(features, W_ln1, b_ln1, W_c1a, W_c1b)
    hf = ha.shape[1] // 2
    pA_lo, pB_lo = _sc_conv_pair(ha[:, :hf], srcA, dstA, wA,
                                 hb[:, :hf], srcB, dstB, wB, stage=True)
    pA_hi, pB_hi = _sc_conv_pair(ha[:, hf:], srcA, dstA, wA,
                                 hb[:, hf:], srcB, dstB, wB, stage=True)
    pA = jnp.concatenate([pA_lo, pA_hi], axis=1)
    pB = jnp.concatenate([pB_lo, pB_hi], axis=1)
    y0, h2a, h2b = _t2(x0, pA, pB, b_c1a + b_c1b, 2.0, bn1_g, bn1_b,
                       W_ln2, b_ln2, W_c2a, W_c2b)
    qA, qB = _sc_conv_pair(h2a, srcA, dstA, wA, h2b, srcB, dstB, wB, stage=True)
    out = _t3(y0, qA, qB, b_c2a + b_c2b, 0.5, bn2_g, bn2_b,
              W_conv.T, b_conv)
    return out


# speedup vs baseline: 2.9868x; 1.0416x over previous
import functools

import jax
import jax.numpy as jnp
from jax import lax
from jax.experimental import pallas as pl
from jax.experimental.pallas import tpu as pltpu
from jax.experimental.pallas import tpu_sc as plsc

NS = 16
LANES = 16
CHUNK = 128


def _t1_body(x_ref, w0_ref, b0_ref, wa_ref, wb_ref,
             o0_ref, oa0_ref, oa1_ref, ob0_ref, ob1_ref):
    x = x_ref[...]
    hf = oa0_ref.shape[1]
    o0_ref[...] = jnp.dot(x, w0_ref[...], preferred_element_type=jnp.float32) + b0_ref[...]
    ya = jnp.dot(x, wa_ref[...], preferred_element_type=jnp.float32)
    yb = jnp.dot(x, wb_ref[...], preferred_element_type=jnp.float32)
    oa0_ref[...] = ya[:, :hf]
    oa1_ref[...] = ya[:, hf:]
    ob0_ref[...] = yb[:, :hf]
    ob1_ref[...] = yb[:, hf:]


def _t1(x, W0, b0, Wa, Wb):
    n, d = x.shape
    h = W0.shape[1]
    blk = 2000
    grid = n // blk
    return pl.pallas_call(
        _t1_body,
        grid=(grid,),
        in_specs=[
            pl.BlockSpec((blk, d), lambda i: (i, 0)),
            pl.BlockSpec((d, h), lambda i: (0, 0)),
            pl.BlockSpec((1, h), lambda i: (0, 0)),
            pl.BlockSpec((d, h), lambda i: (0, 0)),
            pl.BlockSpec((d, h), lambda i: (0, 0)),
        ],
        out_specs=[pl.BlockSpec((blk, h), lambda i: (i, 0))] +
                  [pl.BlockSpec((blk, h // 2), lambda i: (i, 0))] * 4,
        out_shape=[jax.ShapeDtypeStruct((n, h), jnp.float32)] +
                  [jax.ShapeDtypeStruct((n, h // 2), jnp.float32)] * 4,
    )(x, W0, b0.reshape(1, h), Wa, Wb)


def _bn_mm3_body(x0_ref, pa0_ref, pa1_ref, pb0_ref, pb1_ref,
                 bsum_ref, g_ref, bb_ref,
                 w0_ref, b0_ref, wa_ref, wb_ref, scale_ref,
                 o0_ref, oa_ref, ob_ref):
    left = x0_ref[...]
    nrows = left.shape[0]
    rsum = jnp.concatenate([pa0_ref[...] + pb0_ref[...],
                            pa1_ref[...] + pb1_ref[...]], axis=1)
    right = scale_ref[0] * (rsum + bsum_ref[...])
    u = jnp.concatenate([left, right], axis=1)
    mu = jnp.sum(u, axis=0, keepdims=True) / nrows
    xc = u - mu
    var = jnp.sum(xc * xc, axis=0, keepdims=True) / nrows
    z = xc * lax.rsqrt(var + 1e-5) * g_ref[...] + bb_ref[...]
    o0_ref[...] = jnp.dot(z, w0_ref[...], preferred_element_type=jnp.float32) + b0_ref[...]
    oa_ref[...] = jnp.dot(z, wa_ref[...], preferred_element_type=jnp.float32)
    ob_ref[...] = jnp.dot(z, wb_ref[...], preferred_element_type=jnp.float32)


def _t2(x0, pa0, pa1, pb0, pb1, bsum, scale, g, bb, W0, b0, Wa, Wb):
    n, h = x0.shape
    c = W0.shape[1]
    return pl.pallas_call(
        _bn_mm3_body,
        out_shape=[
            jax.ShapeDtypeStruct((n, c), jnp.float32),
            jax.ShapeDtypeStruct((n, c), jnp.float32),
            jax.ShapeDtypeStruct((n, c), jnp.float32),
        ],
    )(x0, pa0, pa1, pb0, pb1, bsum.reshape(1, h), g.reshape(1, 2 * h),
      bb.reshape(1, 2 * h), W0, b0.reshape(1, c), Wa, Wb,
      jnp.asarray([scale], jnp.float32))


def _bn_final_body(x0_ref, pa_ref, pb_ref, bsum_ref, g_ref, bb_ref,
                   wt_ref, bout_ref, scale_ref, o_ref):
    left = x0_ref[...]
    nrows = left.shape[0]
    right = scale_ref[0] * (pa_ref[...][:nrows] + pb_ref[...][:nrows] + bsum_ref[...])
    u = jnp.concatenate([left, right], axis=1)
    mu = jnp.sum(u, axis=0, keepdims=True) / nrows
    xc = u - mu
    var = jnp.sum(xc * xc, axis=0, keepdims=True) / nrows
    z = xc * lax.rsqrt(var + 1e-5) * g_ref[...] + bb_ref[...]
    o_ref[...] = jnp.dot(z, wt_ref[...], preferred_element_type=jnp.float32) + bout_ref[...]


def _t3(x0, pa, pb, bsum, scale, g, bb, Wt, bout):
    n, c = x0.shape
    cout = Wt.shape[1]
    return pl.pallas_call(
        _bn_final_body,
        out_shape=jax.ShapeDtypeStruct((n, cout), jnp.float32),
    )(x0, pa, pb, bsum.reshape(1, c), g.reshape(1, 2 * c), bb.reshape(1, 2 * c),
      Wt, bout.reshape(1, cout), jnp.asarray([scale], jnp.float32))


def _sc_conv_pair(hAs, srcA, dstA, wA, hBs, srcB, dstB, wB):
    nh = len(hAs)
    n, f = hAs[0].shape
    n_chunks = srcA.shape[0] // NS
    rows_per_tile = n // NS
    mesh = plsc.VectorSubcoreMesh(core_axis_name="c", subcore_axis_name="s")

    @functools.partial(
        pl.kernel,
        out_type=tuple(jax.ShapeDtypeStruct((n, f), jnp.float32)
                       for _ in range(2 * nh)),
        mesh=mesh,
        compiler_params=pltpu.CompilerParams(use_tc_tiling_on_sc=False),
        scratch_types=[
            pltpu.VMEM((n_chunks, CHUNK), jnp.int32),
            pltpu.VMEM((n_chunks, CHUNK), jnp.int32),
            pltpu.VMEM((n_chunks * CHUNK,), jnp.float32),
            pltpu.VMEM((CHUNK, f), jnp.float32),
            pltpu.VMEM((CHUNK, f), jnp.float32),
            pltpu.VMEM((CHUNK, f), jnp.float32),
            pltpu.VMEM((CHUNK, f), jnp.float32),
            pltpu.VMEM_SHARED((n, f), jnp.float32),
            pltpu.VMEM_SHARED((n, f), jnp.float32),
        ] + [pltpu.SemaphoreType.DMA] * 8,
    )
    def k(*refs):
        hA_refs = refs[0:nh]
        srcA_ref, dstA_ref, wA_ref = refs[nh:nh + 3]
        hB_refs = refs[nh + 3:2 * nh + 3]
        srcB_ref, dstB_ref, wB_ref = refs[2 * nh + 3:2 * nh + 6]
        o = 2 * nh + 6
        outA_refs = refs[o:o + nh]
        outB_refs = refs[o + nh:o + 2 * nh]
        (src_v, dst_v, w_v, rows0, rows1, rows2, rows3, acc, h_s,
         sg0, sg1, sg2, sg3, ss0, ss1, ss2, ss3) = refs[o + 2 * nh:]
        rows_bufs = (rows0, rows1, rows2, rows3)
        gsems = (sg0, sg1, sg2, sg3)
        ssems = (ss0, ss1, ss2, ss3)
        c = lax.axis_index("c")
        s = lax.axis_index("s")
        zero16 = jnp.zeros((LANES,), jnp.float32)

        def run(h_refs, src_ref, dst_ref, w_ref, out_refs):
            base = s * n_chunks
            pltpu.sync_copy(src_ref.at[pl.ds(base, n_chunks)], src_v)
            pltpu.sync_copy(dst_ref.at[pl.ds(base, n_chunks)], dst_v)
            pltpu.sync_copy(w_ref.at[pl.ds(base * CHUNK, n_chunks * CHUNK)], w_v)
            row0 = s * rows_per_tile

            def gather_start(j, b):
                pltpu.async_copy(h_s.at[src_v.at[j]], rows_bufs[b], gsems[b])

            def gather_wait(b):
                pltpu.make_async_copy(h_s.at[src_v.at[0]], rows_bufs[b],
                                      gsems[b]).wait()

            def scatter_start(j, b):
                pltpu.async_copy(rows_bufs[b], acc.at[dst_v.at[j]], ssems[b],
                                 add=True)

            def scatter_wait(b):
                pltpu.make_async_copy(rows_bufs[b], acc.at[dst_v.at[0]],
                                      ssems[b]).wait()

            def compute(j, rows):
                def group_body(ge, c2):
                    wg = w_v[pl.ds(j * CHUNK + ge * LANES, LANES)]
                    for i in range(LANES):
                        wv = jnp.full((LANES,), wg[i])
                        e = ge * LANES + i
                        for g in range(f // LANES):
                            sl = rows[e, pl.ds(g * LANES, LANES)]
                            rows[e, pl.ds(g * LANES, LANES)] = sl * wv
                    return c2
                lax.fori_loop(0, CHUNK // LANES, group_body, 0)

            for h_ref, out_ref in zip(h_refs, out_refs):
                nfs = rows_per_tile // CHUNK
                for b in range(nfs):
                    off = row0 + b * CHUNK
                    pltpu.sync_copy(h_ref.at[pl.ds(off, CHUNK)], rows0)
                    pltpu.sync_copy(rows0, h_s.at[pl.ds(off, CHUNK)])
                rem_s = rows_per_tile - nfs * CHUNK
                if rem_s:
                    off = row0 + nfs * CHUNK
                    pltpu.sync_copy(h_ref.at[pl.ds(off, rem_s)],
                                    rows0.at[pl.ds(0, rem_s)])
                    pltpu.sync_copy(rows0.at[pl.ds(0, rem_s)],
                                    h_s.at[pl.ds(off, rem_s)])

                def zrow(i, carry):
                    for g in range(f // LANES):
                        rows0[i, pl.ds(g * LANES, LANES)] = zero16
                    return carry
                lax.fori_loop(0, CHUNK, zrow, 0)
                nfull = rows_per_tile // CHUNK
                for blk in range(nfull):
                    pltpu.sync_copy(rows0, acc.at[pl.ds(row0 + blk * CHUNK, CHUNK)])
                rem = rows_per_tile - nfull * CHUNK
                if rem:
                    pltpu.sync_copy(rows0.at[pl.ds(0, rem)],
                                    acc.at[pl.ds(row0 + nfull * CHUNK, rem)])
                plsc.subcore_barrier()

                gather_start(0, 0)
                gather_start(1, 1)

                def quad_body(q, carry):
                    for b in range(4):
                        j = 4 * q + b
                        gather_wait(b)
                        compute(j, rows_bufs[b])
                        scatter_start(j, b)
                        b2 = (b + 2) % 4

                        @pl.when(j + 2 < n_chunks)
                        def _():
                            @pl.when(j >= 2)
                            def _():
                                scatter_wait(b2)
                            gather_start(j + 2, b2)
                    return carry
                lax.fori_loop(0, n_chunks // 4, quad_body, 0)
                for b in range(4):
                    scatter_wait((n_chunks - 4 + b) % 4)
                plsc.subcore_barrier()
                pltpu.sync_copy(acc.at[pl.ds(row0, rows_per_tile)],
                                out_ref.at[pl.ds(row0, rows_per_tile)])

        @pl.when(c == 0)
        def _():
            run(hA_refs, srcA_ref, dstA_ref, wA_ref, outA_refs)

        @pl.when(c == 1)
        def _():
            run(hB_refs, srcB_ref, dstB_ref, wB_ref, outB_refs)

    outs = k(*hAs, srcA, dstA, wA, *hBs, srcB, dstB, wB)
    return outs[:nh], outs[nh:]


def _prep_edges(ei, ew):
    e = ei.shape[1]
    n_chunks = -(-e // (NS * CHUNK))
    n_chunks = -(-n_chunks // 8) * 8
    epad = NS * n_chunks * CHUNK
    pad = epad - e
    src = jnp.concatenate([ei[0], jnp.zeros((pad,), jnp.int32)]).reshape(NS * n_chunks, CHUNK)
    dst = jnp.concatenate([ei[1], jnp.zeros((pad,), jnp.int32)]).reshape(NS * n_chunks, CHUNK)
    w = jnp.concatenate([ew, jnp.zeros((pad,), jnp.float32)])
    return src, dst, w


def kernel(features, edge_index, edge_weight, edge_index2, edge_weight2,
           W_ln1, b_ln1, W_c1a, b_c1a, W_c1b, b_c1b, bn1_g, bn1_b,
           W_ln2, b_ln2, W_c2a, b_c2a, W_c2b, b_c2b, bn2_g, bn2_b,
           W_conv, b_conv):
    srcA, dstA, wA = _prep_edges(edge_index, edge_weight)
    srcB, dstB, wB = _prep_edges(edge_index2, edge_weight2)

    x0, ha0, ha1, hb0, hb1 = _t1(features, W_ln1, b_ln1, W_c1a, W_c1b)
    pAs, pBs = _sc_conv_pair((ha0, ha1), srcA, dstA, wA,
                             (hb0, hb1), srcB, dstB, wB)
    y0, h2a, h2b = _t2(x0, pAs[0], pAs[1], pBs[0], pBs[1],
                       b_c1a + b_c1b, 2.0, bn1_g, bn1_b,
                       W_ln2, b_ln2, W_c2a, W_c2b)
    qAs, qBs = _sc_conv_pair((h2a,), srcA, dstA, wA, (h2b,), srcB, dstB, wB)
    out = _t3(y0, qAs[0], qBs[0], b_c2a + b_c2b, 0.5, bn2_g, bn2_b,
              W_conv.T, b_conv)
    return out


# speedup vs baseline: 3.0342x; 1.0159x over previous
import functools

import jax
import jax.numpy as jnp
from jax import lax
from jax.experimental import pallas as pl
from jax.experimental.pallas import tpu as pltpu
from jax.experimental.pallas import tpu_sc as plsc

NS = 16
LANES = 16
CHUNK = 128
NBUF = 5
LOOK = 3


def _t1_body(x_ref, w0_ref, b0_ref, wa_ref, wb_ref,
             o0_ref, oa0_ref, oa1_ref, ob0_ref, ob1_ref):
    x = x_ref[...]
    hf = oa0_ref.shape[1]
    o0_ref[...] = jnp.dot(x, w0_ref[...], preferred_element_type=jnp.float32) + b0_ref[...]
    ya = jnp.dot(x, wa_ref[...], preferred_element_type=jnp.float32)
    yb = jnp.dot(x, wb_ref[...], preferred_element_type=jnp.float32)
    oa0_ref[...] = ya[:, :hf]
    oa1_ref[...] = ya[:, hf:]
    ob0_ref[...] = yb[:, :hf]
    ob1_ref[...] = yb[:, hf:]


def _t1(x, W0, b0, Wa, Wb):
    n, d = x.shape
    h = W0.shape[1]
    blk = 2000
    grid = n // blk
    return pl.pallas_call(
        _t1_body,
        grid=(grid,),
        in_specs=[
            pl.BlockSpec((blk, d), lambda i: (i, 0)),
            pl.BlockSpec((d, h), lambda i: (0, 0)),
            pl.BlockSpec((1, h), lambda i: (0, 0)),
            pl.BlockSpec((d, h), lambda i: (0, 0)),
            pl.BlockSpec((d, h), lambda i: (0, 0)),
        ],
        out_specs=[pl.BlockSpec((blk, h), lambda i: (i, 0))] +
                  [pl.BlockSpec((blk, h // 2), lambda i: (i, 0))] * 4,
        out_shape=[jax.ShapeDtypeStruct((n, h), jnp.float32)] +
                  [jax.ShapeDtypeStruct((n, h // 2), jnp.float32)] * 4,
    )(x, W0, b0.reshape(1, h), Wa, Wb)


def _bn_mm3_body(x0_ref, pa0_ref, pa1_ref, pb0_ref, pb1_ref,
                 bsum_ref, g_ref, bb_ref,
                 w0_ref, b0_ref, wa_ref, wb_ref, scale_ref,
                 o0_ref, oa_ref, ob_ref):
    left = x0_ref[...]
    nrows = left.shape[0]
    rsum = jnp.concatenate([pa0_ref[...] + pb0_ref[...],
                            pa1_ref[...] + pb1_ref[...]], axis=1)
    right = scale_ref[0] * (rsum + bsum_ref[...])
    u = jnp.concatenate([left, right], axis=1)
    mu = jnp.sum(u, axis=0, keepdims=True) / nrows
    xc = u - mu
    var = jnp.sum(xc * xc, axis=0, keepdims=True) / nrows
    z = xc * lax.rsqrt(var + 1e-5) * g_ref[...] + bb_ref[...]
    o0_ref[...] = jnp.dot(z, w0_ref[...], preferred_element_type=jnp.float32) + b0_ref[...]
    oa_ref[...] = jnp.dot(z, wa_ref[...], preferred_element_type=jnp.float32)
    ob_ref[...] = jnp.dot(z, wb_ref[...], preferred_element_type=jnp.float32)


def _t2(x0, pa0, pa1, pb0, pb1, bsum, scale, g, bb, W0, b0, Wa, Wb):
    n, h = x0.shape
    c = W0.shape[1]
    return pl.pallas_call(
        _bn_mm3_body,
        out_shape=[
            jax.ShapeDtypeStruct((n, c), jnp.float32),
            jax.ShapeDtypeStruct((n, c), jnp.float32),
            jax.ShapeDtypeStruct((n, c), jnp.float32),
        ],
    )(x0, pa0, pa1, pb0, pb1, bsum.reshape(1, h), g.reshape(1, 2 * h),
      bb.reshape(1, 2 * h), W0, b0.reshape(1, c), Wa, Wb,
      jnp.asarray([scale], jnp.float32))


def _bn_final_body(x0_ref, pa_ref, pb_ref, bsum_ref, g_ref, bb_ref,
                   wt_ref, bout_ref, scale_ref, o_ref):
    left = x0_ref[...]
    nrows = left.shape[0]
    right = scale_ref[0] * (pa_ref[...][:nrows] + pb_ref[...][:nrows] + bsum_ref[...])
    u = jnp.concatenate([left, right], axis=1)
    mu = jnp.sum(u, axis=0, keepdims=True) / nrows
    xc = u - mu
    var = jnp.sum(xc * xc, axis=0, keepdims=True) / nrows
    z = xc * lax.rsqrt(var + 1e-5) * g_ref[...] + bb_ref[...]
    o_ref[...] = jnp.dot(z, wt_ref[...], preferred_element_type=jnp.float32) + bout_ref[...]


def _t3(x0, pa, pb, bsum, scale, g, bb, Wt, bout):
    n, c = x0.shape
    cout = Wt.shape[1]
    return pl.pallas_call(
        _bn_final_body,
        out_shape=jax.ShapeDtypeStruct((n, cout), jnp.float32),
    )(x0, pa, pb, bsum.reshape(1, c), g.reshape(1, 2 * c), bb.reshape(1, 2 * c),
      Wt, bout.reshape(1, cout), jnp.asarray([scale], jnp.float32))


def _sc_conv_pair(hAs, srcA, dstA, wA, hBs, srcB, dstB, wB):
    nh = len(hAs)
    n, f = hAs[0].shape
    n_chunks = srcA.shape[0] // NS
    rows_per_tile = n // NS
    mesh = plsc.VectorSubcoreMesh(core_axis_name="c", subcore_axis_name="s")

    @functools.partial(
        pl.kernel,
        out_type=tuple(jax.ShapeDtypeStruct((n, f), jnp.float32)
                       for _ in range(2 * nh)),
        mesh=mesh,
        compiler_params=pltpu.CompilerParams(use_tc_tiling_on_sc=False),
        scratch_types=[
            pltpu.VMEM((n_chunks, CHUNK), jnp.int32),
            pltpu.VMEM((n_chunks, CHUNK), jnp.int32),
            pltpu.VMEM((n_chunks * CHUNK,), jnp.float32),
        ] + [pltpu.VMEM((CHUNK, f), jnp.float32)] * NBUF + [
            pltpu.VMEM_SHARED((n, f), jnp.float32),
            pltpu.VMEM_SHARED((n, f), jnp.float32),
        ] + [pltpu.SemaphoreType.DMA] * (2 * NBUF),
    )
    def k(*refs):
        hA_refs = refs[0:nh]
        srcA_ref, dstA_ref, wA_ref = refs[nh:nh + 3]
        hB_refs = refs[nh + 3:2 * nh + 3]
        srcB_ref, dstB_ref, wB_ref = refs[2 * nh + 3:2 * nh + 6]
        o = 2 * nh + 6
        outA_refs = refs[o:o + nh]
        outB_refs = refs[o + nh:o + 2 * nh]
        scr = refs[o + 2 * nh:]
        src_v, dst_v, w_v = scr[0:3]
        rows_bufs = scr[3:3 + NBUF]
        acc, h_s = scr[3 + NBUF:5 + NBUF]
        gsems = scr[5 + NBUF:5 + 2 * NBUF]
        ssems = scr[5 + 2 * NBUF:5 + 3 * NBUF]
        rows0 = rows_bufs[0]
        c = lax.axis_index("c")
        s = lax.axis_index("s")
        zero16 = jnp.zeros((LANES,), jnp.float32)

        def run(h_refs, src_ref, dst_ref, w_ref, out_refs):
            base = s * n_chunks
            pltpu.sync_copy(src_ref.at[pl.ds(base, n_chunks)], src_v)
            pltpu.sync_copy(dst_ref.at[pl.ds(base, n_chunks)], dst_v)
            pltpu.sync_copy(w_ref.at[pl.ds(base * CHUNK, n_chunks * CHUNK)], w_v)
            row0 = s * rows_per_tile

            def gather_start(j, b):
                pltpu.async_copy(h_s.at[src_v.at[j]], rows_bufs[b], gsems[b])

            def gather_wait(b):
                pltpu.make_async_copy(h_s.at[src_v.at[0]], rows_bufs[b],
                                      gsems[b]).wait()

            def scatter_start(j, b):
                pltpu.async_copy(rows_bufs[b], acc.at[dst_v.at[j]], ssems[b],
                                 add=True)

            def scatter_wait(b):
                pltpu.make_async_copy(rows_bufs[b], acc.at[dst_v.at[0]],
                                      ssems[b]).wait()

            def compute(j, rows):
                def group_body(ge, c2):
                    wg = w_v[pl.ds(j * CHUNK + ge * LANES, LANES)]
                    for i in range(LANES):
                        wv = jnp.full((LANES,), wg[i])
                        e = ge * LANES + i
                        for g in range(f // LANES):
                            sl = rows[e, pl.ds(g * LANES, LANES)]
                            rows[e, pl.ds(g * LANES, LANES)] = sl * wv
                    return c2
                lax.fori_loop(0, CHUNK // LANES, group_body, 0)

            for h_ref, out_ref in zip(h_refs, out_refs):
                nfs = rows_per_tile // CHUNK
                for b in range(nfs):
                    off = row0 + b * CHUNK
                    pltpu.sync_copy(h_ref.at[pl.ds(off, CHUNK)], rows0)
                    pltpu.sync_copy(rows0, h_s.at[pl.ds(off, CHUNK)])
                rem_s = rows_per_tile - nfs * CHUNK
                if rem_s:
                    off = row0 + nfs * CHUNK
                    pltpu.sync_copy(h_ref.at[pl.ds(off, rem_s)],
                                    rows0.at[pl.ds(0, rem_s)])
                    pltpu.sync_copy(rows0.at[pl.ds(0, rem_s)],
                                    h_s.at[pl.ds(off, rem_s)])

                def zrow(i, carry):
                    for g in range(f // LANES):
                        rows0[i, pl.ds(g * LANES, LANES)] = zero16
                    return carry
                lax.fori_loop(0, CHUNK, zrow, 0)
                nfull = rows_per_tile // CHUNK
                for blk in range(nfull):
                    pltpu.sync_copy(rows0, acc.at[pl.ds(row0 + blk * CHUNK, CHUNK)])
                rem = rows_per_tile - nfull * CHUNK
                if rem:
                    pltpu.sync_copy(rows0.at[pl.ds(0, rem)],
                                    acc.at[pl.ds(row0 + nfull * CHUNK, rem)])
                plsc.subcore_barrier()

                for jj in range(LOOK):
                    gather_start(jj, jj)

                def ring_body(q, carry):
                    for b in range(NBUF):
                        j = NBUF * q + b
                        gather_wait(b)
                        compute(j, rows_bufs[b])
                        scatter_start(j, b)
                        b2 = (b + LOOK) % NBUF

                        @pl.when(j + LOOK < n_chunks)
                        def _():
                            @pl.when(j >= NBUF - LOOK)
                            def _():
                                scatter_wait(b2)
                            gather_start(j + LOOK, b2)
                    return carry
                lax.fori_loop(0, n_chunks // NBUF, ring_body, 0)
                for b in range(NBUF):
                    scatter_wait((n_chunks - NBUF + b) % NBUF)
                plsc.subcore_barrier()
                pltpu.sync_copy(acc.at[pl.ds(row0, rows_per_tile)],
                                out_ref.at[pl.ds(row0, rows_per_tile)])

        @pl.when(c == 0)
        def _():
            run(hA_refs, srcA_ref, dstA_ref, wA_ref, outA_refs)

        @pl.when(c == 1)
        def _():
            run(hB_refs, srcB_ref, dstB_ref, wB_ref, outB_refs)

    outs = k(*hAs, srcA, dstA, wA, *hBs, srcB, dstB, wB)
    return outs[:nh], outs[nh:]


def _prep_edges(ei, ew):
    e = ei.shape[1]
    n_chunks = -(-e // (NS * CHUNK))
    n_chunks = -(-n_chunks // 8) * 8
    epad = NS * n_chunks * CHUNK
    pad = epad - e
    src = jnp.concatenate([ei[0], jnp.zeros((pad,), jnp.int32)]).reshape(NS * n_chunks, CHUNK)
    dst = jnp.concatenate([ei[1], jnp.zeros((pad,), jnp.int32)]).reshape(NS * n_chunks, CHUNK)
    w = jnp.concatenate([ew, jnp.zeros((pad,), jnp.float32)])
    return src, dst, w


def kernel(features, edge_index, edge_weight, edge_index2, edge_weight2,
           W_ln1, b_ln1, W_c1a, b_c1a, W_c1b, b_c1b, bn1_g, bn1_b,
           W_ln2, b_ln2, W_c2a, b_c2a, W_c2b, b_c2b, bn2_g, bn2_b,
           W_conv, b_conv):
    srcA, dstA, wA = _prep_edges(edge_index, edge_weight)
    srcB, dstB, wB = _prep_edges(edge_index2, edge_weight2)

    x0, ha0, ha1, hb0, hb1 = _t1(features, W_ln1, b_ln1, W_c1a, W_c1b)
    pAs, pBs = _sc_conv_pair((ha0, ha1), srcA, dstA, wA,
                             (hb0, hb1), srcB, dstB, wB)
    y0, h2a, h2b = _t2(x0, pAs[0], pAs[1], pBs[0], pBs[1],
                       b_c1a + b_c1b, 2.0, bn1_g, bn1_b,
                       W_ln2, b_ln2, W_c2a, W_c2b)
    qAs, qBs = _sc_conv_pair((h2a,), srcA, dstA, wA, (h2b,), srcB, dstB, wB)
    out = _t3(y0, qAs[0], qBs[0], b_c2a + b_c2b, 0.5, bn2_g, bn2_b,
              W_conv.T, b_conv)
    return out
